# Initial kernel scaffold; baseline (speedup 1.0000x reference)
import jax
import jax.numpy as jnp
from jax.experimental import pallas as pl


def kernel(x, edge_index, edge_attr, W, b, gamma, beta):
    raise NotImplementedError("write your pallas kernel here")


# speedup vs baseline: 23.5989x; 23.5989x over previous
import functools

import jax
import jax.numpy as jnp
from jax import lax
from jax.experimental import pallas as pl
from jax.experimental.pallas import tpu as pltpu
from jax.experimental.pallas import tpu_sc as plsc

N = 10000
D = 128
NC, NS, LANES = 2, 16, 16
NW = NC * NS
K = 128
NT = 10240
NPAD = 16
MB = 1000
GRID = N // MB

_mesh = plsc.VectorSubcoreMesh(core_axis_name="c", subcore_axis_name="s")


def _deg_kernel(nb):
    rows_per_tile = NT // NS

    @functools.partial(
        pl.kernel,
        out_type=jax.ShapeDtypeStruct((NC, NT, D), jnp.float32),
        mesh=_mesh,
        scratch_types=[
            pltpu.VMEM((nb, K), jnp.int32),
            pltpu.VMEM((K, D), jnp.float32),
            pltpu.VMEM_SHARED((NT, D), jnp.float32),
        ],
    )
    def deg(col_hbm, z_hbm, o_hbm, deg_hbm, colv, ones, deg_sp):
        cid = lax.axis_index("c")
        sid = lax.axis_index("s")
        wid = cid * NS + sid
        pltpu.sync_copy(col_hbm.at[wid], colv)
        pltpu.sync_copy(o_hbm, ones)
        pltpu.sync_copy(
            z_hbm.at[pl.ds(sid * rows_per_tile, rows_per_tile)],
            deg_sp.at[pl.ds(sid * rows_per_tile, rows_per_tile)],
        )
        plsc.subcore_barrier()

        def body(b, carry):
            pltpu.sync_copy(ones, deg_sp.at[colv.at[b]], add=True)
            return carry

        lax.fori_loop(0, nb, body, 0)
        plsc.subcore_barrier()
        pltpu.sync_copy(
            deg_sp.at[pl.ds(sid * rows_per_tile, rows_per_tile)],
            deg_hbm.at[cid, pl.ds(sid * rows_per_tile, rows_per_tile)],
        )

    return deg


def _agg_kernel(nb):
    zrows = NT // NS

    @functools.partial(
        pl.kernel,
        out_type=jax.ShapeDtypeStruct((NC, NT, D), jnp.float32),
        mesh=_mesh,
        scratch_types=[
            pltpu.VMEM((nb, K), jnp.int32),
            pltpu.VMEM((nb, K), jnp.int32),
            pltpu.VMEM((K, D), jnp.float32),
            pltpu.VMEM_SHARED((NT, D), jnp.float32),
        ],
    )
    def agg(g_hbm, row_hbm, col_hbm, z_hbm, agg_hbm, rowv, colv, buf, agg_sp):
        cid = lax.axis_index("c")
        sid = lax.axis_index("s")
        wid = cid * NS + sid
        pltpu.sync_copy(row_hbm.at[wid], rowv)
        pltpu.sync_copy(col_hbm.at[wid], colv)
        pltpu.sync_copy(
            z_hbm.at[pl.ds(sid * zrows, zrows)],
            agg_sp.at[pl.ds(sid * zrows, zrows)],
        )
        plsc.subcore_barrier()

        def body(b, carry):
            pltpu.sync_copy(g_hbm.at[rowv.at[b]], buf)
            pltpu.sync_copy(buf, agg_sp.at[colv.at[b]], add=True)
            return carry

        lax.fori_loop(0, nb, body, 0)
        plsc.subcore_barrier()
        pltpu.sync_copy(
            agg_sp.at[pl.ds(sid * zrows, zrows)],
            agg_hbm.at[cid, pl.ds(sid * zrows, zrows)],
        )

    return agg


def _linear_body(x_ref, w_ref, degp_ref, g_ref, dis_ref):
    deg = degp_ref[0, :, 0:1] + degp_ref[1, :, 0:1]
    dis = lax.rsqrt(deg)
    h = lax.dot_general(
        x_ref[...], w_ref[...], (((1,), (1,)), ((), ())),
        preferred_element_type=jnp.float32,
        precision=lax.Precision.HIGHEST,
    )
    g_ref[...] = h * dis
    dis_ref[...] = dis


def _finale1_body(aggp_ref, dis_ref, b_ref, y_ref, stats_ref):
    a = aggp_ref[0] + aggp_ref[1]
    y = jnp.maximum(a * dis_ref[...] + b_ref[...], 0.0)
    y_ref[...] = y

    @pl.when(pl.program_id(0) == 0)
    def _():
        stats_ref[...] = jnp.zeros((2, D), jnp.float32)

    stats_ref[...] += jnp.concatenate(
        [jnp.sum(y, 0, keepdims=True), jnp.sum(y * y, 0, keepdims=True)], axis=0
    )


def _finale2_body(y_ref, stats_ref, gamma_ref, beta_ref, out_ref):
    inv_n = jnp.float32(1.0 / N)
    mean = stats_ref[0:1, :] * inv_n
    ex2 = stats_ref[1:2, :] * inv_n
    var = ex2 - mean * mean
    scale = gamma_ref[...] * lax.rsqrt(var + 1e-5)
    shift = beta_ref[...] - mean * scale
    out_ref[...] = y_ref[...] * scale + shift


def kernel(x, edge_index, edge_attr, W, b, gamma, beta):
    del edge_attr
    ei = edge_index.astype(jnp.int32)
    loops = jnp.arange(N, dtype=jnp.int32)
    row = jnp.concatenate([ei[0], loops])
    col = jnp.concatenate([ei[1], loops])
    e = row.shape[0]
    nb = -(-e // (NW * K))
    pad = NW * nb * K - e
    pr = (jnp.arange(pad, dtype=jnp.int32) * 7919) % N
    pc = N + (jnp.arange(pad, dtype=jnp.int32) % NPAD)
    row3 = jnp.concatenate([row, pr]).reshape(NW, nb, K)
    col3 = jnp.concatenate([col, pc]).reshape(NW, nb, K)

    degp = _deg_kernel(nb)(
        col3,
        jnp.zeros((NT, D), jnp.float32),
        jnp.ones((K, D), jnp.float32),
    )

    g, dis = pl.pallas_call(
        _linear_body,
        grid=(GRID,),
        in_specs=[
            pl.BlockSpec((MB, D), lambda i: (i, 0)),
            pl.BlockSpec((D, D), lambda i: (0, 0)),
            pl.BlockSpec((2, MB, D), lambda i: (0, i, 0)),
        ],
        out_specs=[
            pl.BlockSpec((MB, D), lambda i: (i, 0)),
            pl.BlockSpec((MB, 1), lambda i: (i, 0)),
        ],
        out_shape=[
            jax.ShapeDtypeStruct((N, D), jnp.float32),
            jax.ShapeDtypeStruct((N, 1), jnp.float32),
        ],
    )(x, W, degp)

    aggp = _agg_kernel(nb)(g, row3, col3, jnp.zeros((NT, D), jnp.float32))

    y, stats = pl.pallas_call(
        _finale1_body,
        grid=(GRID,),
        in_specs=[
            pl.BlockSpec((2, MB, D), lambda i: (0, i, 0)),
            pl.BlockSpec((MB, 1), lambda i: (i, 0)),
            pl.BlockSpec((1, D), lambda i: (0, 0)),
        ],
        out_specs=[
            pl.BlockSpec((MB, D), lambda i: (i, 0)),
            pl.BlockSpec((2, D), lambda i: (0, 0)),
        ],
        out_shape=[
            jax.ShapeDtypeStruct((N, D), jnp.float32),
            jax.ShapeDtypeStruct((2, D), jnp.float32),
        ],
    )(aggp, dis, b.reshape(1, D))

    out = pl.pallas_call(
        _finale2_body,
        grid=(GRID,),
        in_specs=[
            pl.BlockSpec((MB, D), lambda i: (i, 0)),
            pl.BlockSpec((2, D), lambda i: (0, 0)),
            pl.BlockSpec((1, D), lambda i: (0, 0)),
            pl.BlockSpec((1, D), lambda i: (0, 0)),
        ],
        out_specs=pl.BlockSpec((MB, D), lambda i: (i, 0)),
        out_shape=jax.ShapeDtypeStruct((N, D), jnp.float32),
    )(y, stats, gamma.reshape(1, D), beta.reshape(1, D))

    return out


# speedup vs baseline: 35.8476x; 1.5190x over previous
import functools

import jax
import jax.numpy as jnp
from jax import lax
from jax.experimental import pallas as pl
from jax.experimental.pallas import tpu as pltpu
from jax.experimental.pallas import tpu_sc as plsc

N = 10000
D = 128
NC, NS, LANES = 2, 16, 16
NW = NC * NS
K = 128
NT = 10112
NPAD = 16
NBUF = 2
CH = 42
MB = 1000
GRID = N // MB
ZR = NT // NS

_mesh = plsc.VectorSubcoreMesh(core_axis_name="c", subcore_axis_name="s")


NTD = 10240
ZRD = NTD // NS


def _deg_kernel(nb):
    ne = nb * K

    @functools.partial(
        pl.kernel,
        out_type=jax.ShapeDtypeStruct((NC * NTD,), jnp.float32),
        mesh=_mesh,
        compiler_params=pltpu.CompilerParams(needs_layout_passes=False),
        scratch_types=[
            pltpu.VMEM((ne,), jnp.int32),
            pltpu.VMEM((NTD,), jnp.float32),
            pltpu.VMEM((NS * ZRD,), jnp.float32),
            pltpu.VMEM((ZRD,), jnp.float32),
            pltpu.VMEM_SHARED((NS * NTD,), jnp.float32),
        ],
    )
    def deg(col_hbm, deg_hbm, colv, hist, redv, outrow, sh):
        cid = lax.axis_index("c")
        sid = lax.axis_index("s")
        wid = cid * NS + sid
        pltpu.sync_copy(col_hbm.at[pl.ds(wid * ne, ne)], colv)
        zvec = jnp.zeros((LANES,), jnp.float32)
        for i in range(NTD // LANES):
            hist[pl.ds(i * LANES, LANES)] = zvec
        ones = jnp.ones((LANES,), jnp.float32)

        def body(b, carry):
            base = b * K
            for j in range(K // LANES):
                v = colv[pl.ds(base + j * LANES, LANES)]
                plsc.addupdate_scatter(hist, [v], ones)
            return carry

        lax.fori_loop(0, nb, body, 0)
        pltpu.sync_copy(hist, sh.at[pl.ds(sid * NTD, NTD)])
        plsc.subcore_barrier()
        for j in range(NS):
            pltpu.sync_copy(
                sh.at[pl.ds(j * NTD + sid * ZRD, ZRD)],
                redv.at[pl.ds(j * ZRD, ZRD)],
            )
        for c in range(ZRD // LANES):
            acc = redv[pl.ds(c * LANES, LANES)]
            for j in range(1, NS):
                acc = acc + redv[pl.ds(j * ZRD + c * LANES, LANES)]
            outrow[pl.ds(c * LANES, LANES)] = acc
        pltpu.sync_copy(
            outrow, deg_hbm.at[pl.ds(cid * NTD + sid * ZRD, ZRD)]
        )

    return deg


def _agg_kernel(nb):
    nchunks = nb // CH

    @functools.partial(
        pl.kernel,
        out_type=jax.ShapeDtypeStruct((NC, NT, D), jnp.float32),
        mesh=_mesh,
        scratch_types=[
            pltpu.VMEM((CH, K), jnp.int32),
            pltpu.VMEM((CH, K), jnp.int32),
            pltpu.VMEM((NBUF, K, D), jnp.float32),
            pltpu.VMEM_SHARED((NT, D), jnp.float32),
            pltpu.SemaphoreType.DMA((NBUF,)),
        ],
    )
    def agg(g_hbm, row_hbm, col_hbm, z_hbm, agg_hbm, rbuf, cbuf, bufs, agg_sp, gs):
        cid = lax.axis_index("c")
        sid = lax.axis_index("s")
        wid = cid * NS + sid
        pltpu.sync_copy(
            z_hbm.at[pl.ds(sid * ZR, ZR)], agg_sp.at[pl.ds(sid * ZR, ZR)]
        )
        plsc.subcore_barrier()

        def start_gather(b, j):
            pltpu.async_copy(g_hbm.at[rbuf.at[b]], bufs.at[j], gs.at[j])

        def wait_gather(j):
            pltpu.make_async_copy(g_hbm.at[rbuf.at[0]], bufs.at[j], gs.at[j]).wait()

        for ch in range(nchunks):
            pltpu.sync_copy(row_hbm.at[wid, ch], rbuf)
            pltpu.sync_copy(col_hbm.at[wid, ch], cbuf)
            for j in range(NBUF):
                start_gather(j, j)

            def inner(oc, carry):
                for jj in range(NBUF):
                    b = oc * NBUF + jj
                    wait_gather(jj)
                    pltpu.sync_copy(bufs.at[jj], agg_sp.at[cbuf.at[b]], add=True)
                    bn = b + NBUF

                    @pl.when(bn < CH)
                    def _():
                        start_gather(bn, jj)

                return carry

            lax.fori_loop(0, CH // NBUF, inner, 0)

        plsc.subcore_barrier()
        pltpu.sync_copy(
            agg_sp.at[pl.ds(sid * ZR, ZR)],
            agg_hbm.at[cid, pl.ds(sid * ZR, ZR)],
        )

    return agg


def _linear_body(x_ref, w_ref, degp_ref, g_ref, dis_ref):
    deg = degp_ref[0] + degp_ref[1]
    dis = lax.rsqrt(deg)
    h = lax.dot_general(
        x_ref[...], w_ref[...], (((1,), (1,)), ((), ())),
        preferred_element_type=jnp.float32,
        precision=lax.Precision.HIGHEST,
    )
    g_ref[...] = h * dis
    dis_ref[...] = dis


def _finale1_body(aggp_ref, dis_ref, b_ref, y_ref, stats_ref):
    a = aggp_ref[0] + aggp_ref[1]
    y = jnp.maximum(a * dis_ref[...] + b_ref[...], 0.0)
    y_ref[...] = y

    @pl.when(pl.program_id(0) == 0)
    def _():
        stats_ref[...] = jnp.zeros((2, D), jnp.float32)

    stats_ref[...] += jnp.concatenate(
        [jnp.sum(y, 0, keepdims=True), jnp.sum(y * y, 0, keepdims=True)], axis=0
    )


def _finale2_body(y_ref, stats_ref, gamma_ref, beta_ref, out_ref):
    inv_n = jnp.float32(1.0 / N)
    mean = stats_ref[0:1, :] * inv_n
    ex2 = stats_ref[1:2, :] * inv_n
    var = ex2 - mean * mean
    scale = gamma_ref[...] * lax.rsqrt(var + 1e-5)
    shift = beta_ref[...] - mean * scale
    out_ref[...] = y_ref[...] * scale + shift


def kernel(x, edge_index, edge_attr, W, b, gamma, beta):
    del edge_attr
    ei = edge_index.astype(jnp.int32)
    loops = jnp.arange(N, dtype=jnp.int32)
    row = jnp.concatenate([ei[0], loops])
    col = jnp.concatenate([ei[1], loops])
    e = row.shape[0]
    nb = -(-(-(-e // (NW * K))) // CH) * CH
    pad = NW * nb * K - e
    pr = (jnp.arange(pad, dtype=jnp.int32) * 7919) % N
    pc = N + (jnp.arange(pad, dtype=jnp.int32) % NPAD)
    row4 = jnp.concatenate([row, pr]).reshape(NW, nb // CH, CH, K)
    col4 = jnp.concatenate([col, pc]).reshape(NW, nb // CH, CH, K)
    col2 = col4.reshape(NW * nb * K)

    degp = _deg_kernel(nb)(col2).reshape(NC, NTD)

    g, dis = pl.pallas_call(
        _linear_body,
        grid=(GRID,),
        in_specs=[
            pl.BlockSpec((MB, D), lambda i: (i, 0)),
            pl.BlockSpec((D, D), lambda i: (0, 0)),
            pl.BlockSpec((2, MB, 1), lambda i: (0, i, 0)),
        ],
        out_specs=[
            pl.BlockSpec((MB, D), lambda i: (i, 0)),
            pl.BlockSpec((MB, 1), lambda i: (i, 0)),
        ],
        out_shape=[
            jax.ShapeDtypeStruct((N, D), jnp.float32),
            jax.ShapeDtypeStruct((N, 1), jnp.float32),
        ],
    )(x, W, degp.reshape(NC, NTD, 1))

    aggp = _agg_kernel(nb)(g, row4, col4, jnp.zeros((NT, D), jnp.float32))

    y, stats = pl.pallas_call(
        _finale1_body,
        grid=(GRID,),
        in_specs=[
            pl.BlockSpec((2, MB, D), lambda i: (0, i, 0)),
            pl.BlockSpec((MB, 1), lambda i: (i, 0)),
            pl.BlockSpec((1, D), lambda i: (0, 0)),
        ],
        out_specs=[
            pl.BlockSpec((MB, D), lambda i: (i, 0)),
            pl.BlockSpec((2, D), lambda i: (0, 0)),
        ],
        out_shape=[
            jax.ShapeDtypeStruct((N, D), jnp.float32),
            jax.ShapeDtypeStruct((2, D), jnp.float32),
        ],
    )(aggp, dis, b.reshape(1, D))

    out = pl.pallas_call(
        _finale2_body,
        grid=(GRID,),
        in_specs=[
            pl.BlockSpec((MB, D), lambda i: (i, 0)),
            pl.BlockSpec((2, D), lambda i: (0, 0)),
            pl.BlockSpec((1, D), lambda i: (0, 0)),
            pl.BlockSpec((1, D), lambda i: (0, 0)),
        ],
        out_specs=pl.BlockSpec((MB, D), lambda i: (i, 0)),
        out_shape=jax.ShapeDtypeStruct((N, D), jnp.float32),
    )(y, stats, gamma.reshape(1, D), beta.reshape(1, D))

    return out
